# baseline (speedup 1.0000x reference)
import functools
import math

import jax
import jax.numpy as jnp
from jax import lax
from jax.experimental import pallas as pl
from jax.experimental.pallas import tpu as pltpu
from jax.experimental.pallas import tpu_sc as plsc

D_MODEL = 64
SCALE = math.sqrt(D_MODEL)
NUM_CORES = 2
NUM_SUBCORES = 16
NUM_WORKERS = NUM_CORES * NUM_SUBCORES
LANES = 16
CHUNK = 128


def _prep_body(n_blocks, tail_off, tail_n, tT_hbm, tail_hbm, tlin_hbm,
               tb0, tb1, tb2, tb3, to0, to1, to2, to3, tlv,
               gsem0, gsem1, gsem2, gsem3, ssem0, ssem1, ssem2, ssem3):
    tbuf = (tb0, tb1, tb2, tb3)
    tout = (to0, to1, to2, to3)
    gsem = (gsem0, gsem1, gsem2, gsem3)
    ssem = (ssem0, ssem1, ssem2, ssem3)

    wid = lax.axis_index("s") * NUM_CORES + lax.axis_index("c")
    nb_w = n_blocks // NUM_WORKERS
    n_left = n_blocks - nb_w * NUM_WORKERS
    base = wid * nb_w

    iota = lax.broadcasted_iota(jnp.int32, (LANES,), 0)
    lane_mask = jnp.full((LANES,), LANES - 1, jnp.int32)

    def start_load(vb, par):
        pltpu.async_copy(
            tT_hbm.at[:, pl.ds(vb * CHUNK, CHUNK)], tbuf[par], gsem[par]
        )

    def wait_load(par):
        pltpu.make_async_copy(
            tT_hbm.at[:, pl.ds(0, CHUNK)], tbuf[par], gsem[par]
        ).wait()

    def start_store(vb, par):
        pltpu.async_copy(
            tout[par], tlin_hbm.at[pl.ds(vb * CHUNK * D_MODEL,
                                         CHUNK * D_MODEL)], ssem[par]
        )

    def wait_store(par):
        pltpu.make_async_copy(
            tout[par], tlin_hbm.at[pl.ds(0, CHUNK * D_MODEL)], ssem[par]
        ).wait()

    def transpose(par):
        @plsc.parallel_loop(0, LANES, unroll=8)
        def _(d):
            rot_d = (iota + d) & lane_mask
            dout_d = rot_d * D_MODEL + iota
            for f0b in range(D_MODEL // LANES):
                rowv = iota + (f0b * LANES)
                for j0b in range(CHUNK // LANES):
                    colv = rot_d + (j0b * LANES)
                    vals = plsc.load_gather(tbuf[par], [rowv, colv])
                    idx_out = dout_d + (j0b * LANES * D_MODEL + f0b * LANES)
                    plsc.store_scatter(tout[par], [idx_out], vals)

    for p in range(4):
        start_load(base + p, p)

    @pl.loop(0, nb_w, step=4)
    def _(c):
        for p in range(4):
            cc = c + p
            wait_load(p)

            @pl.when(cc >= 4)
            def _():
                wait_store(p)

            transpose(p)
            start_store(base + cc, p)

            @pl.when(cc + 4 < nb_w)
            def _():
                start_load(base + cc + 4, p)

    for p in range(4):
        wait_store(p)

    @pl.when(wid < n_left)
    def _():
        vb = nb_w * NUM_WORKERS + wid
        pltpu.sync_copy(tT_hbm.at[:, pl.ds(vb * CHUNK, CHUNK)], tbuf[0])
        transpose(0)
        start_store(vb, 0)
        wait_store(0)

    @pl.when(wid == n_left)
    def _():
        pltpu.sync_copy(tail_hbm, tlv)
        pltpu.sync_copy(
            tlv, tlin_hbm.at[pl.ds(tail_off * D_MODEL, tail_n * D_MODEL)]
        )


def _emb_body(n_sb, x4_hbm, tp_hbm, out_hbm, idx_v, r0, r1, r2, r3,
              b0, b1, b2, b3,
              gsem0, gsem1, gsem2, gsem3, ssem0, ssem1, ssem2, ssem3):
    rbuf = (r0, r1, r2, r3)
    bbuf = (b0, b1, b2, b3)
    gsem = (gsem0, gsem1, gsem2, gsem3)
    ssem = (ssem0, ssem1, ssem2, ssem3)

    wid = lax.axis_index("s") * NUM_CORES + lax.axis_index("c")
    pltpu.sync_copy(x4_hbm.at[:, wid], idx_v)

    iota = lax.broadcasted_iota(jnp.int32, (LANES,), 0)

    def start_gather(c, par):
        pltpu.async_copy(
            tp_hbm.at[idx_v.at[c // 8, c % 8]], rbuf[par], gsem[par]
        )

    def wait_gather(par):
        pltpu.make_async_copy(
            tp_hbm.at[idx_v.at[0, 0]], rbuf[par], gsem[par]
        ).wait()

    def start_store(c, par):
        pltpu.async_copy(
            bbuf[par].at[:, :, pl.ds(0, CHUNK)], out_hbm.at[c, :, wid],
            ssem[par],
        )

    def wait_store(par):
        pltpu.make_async_copy(
            bbuf[par].at[:, :, pl.ds(0, CHUNK)], out_hbm.at[0, :, wid],
            ssem[par],
        ).wait()

    fr_vec = lax.rem(iota, jnp.full((LANES,), 8, jnp.int32))
    fb_vecs = [
        lax.div(iota, jnp.full((LANES,), 8, jnp.int32)) + (2 * g)
        for g in range(4)
    ]

    def transpose_scale(par):
        @plsc.parallel_loop(0, CHUNK, unroll=4)
        def _(j):
            jv = jnp.zeros((LANES,), jnp.int32) + j
            for g in range(4):
                vals = rbuf[par][j, pl.ds(g * LANES, LANES)] * SCALE
                plsc.store_scatter(bbuf[par], [fb_vecs[g], fr_vec, jv], vals)

    n_chunks = n_sb * 8
    for p in range(4):
        start_gather(p, p)

    @pl.loop(0, n_chunks, step=4)
    def _(c):
        for p in range(4):
            cc = c + p
            wait_gather(p)

            @pl.when(cc >= 4)
            def _():
                wait_store(p)

            transpose_scale(p)
            start_store(cc, p)

            @pl.when(cc + 4 < n_chunks)
            def _():
                start_gather(cc + 4, p)

    for p in range(4):
        wait_store(p)


def kernel(x, table):
    batch, seq = x.shape
    vocab, d_model = table.shape
    assert d_model == D_MODEL and seq % 8 == 0 and batch % CHUNK == 0
    n_sb = seq // 8
    n_wb = batch // CHUNK
    assert n_wb == NUM_WORKERS

    x4 = x.T.reshape(n_sb, 8, n_wb, CHUNK).transpose(0, 2, 1, 3)

    mesh = plsc.VectorSubcoreMesh(
        core_axis_name="c",
        subcore_axis_name="s",
        num_cores=NUM_CORES,
        num_subcores=NUM_SUBCORES,
    )

    n_blocks = vocab // CHUNK
    tail_off = n_blocks * CHUNK
    tail_n = vocab - tail_off
    tT = table.T
    tail = table[tail_off:].reshape(tail_n * D_MODEL)
    prep_fn = pl.kernel(
        functools.partial(_prep_body, n_blocks, tail_off, tail_n),
        out_type=jax.ShapeDtypeStruct((vocab * D_MODEL,), jnp.float32),
        mesh=mesh,
        scratch_types=(
            [pltpu.VMEM((D_MODEL, CHUNK), jnp.float32)] * 4
            + [pltpu.VMEM((CHUNK * D_MODEL,), jnp.float32)] * 4
            + [pltpu.VMEM((tail_n * D_MODEL,), jnp.float32)]
            + [pltpu.SemaphoreType.DMA] * 8
        ),
        compiler_params=pltpu.CompilerParams(needs_layout_passes=False),
    )
    tlin = prep_fn(tT, tail).reshape(vocab, D_MODEL)
    grid_fn = pl.kernel(
        functools.partial(_emb_body, n_sb),
        out_type=jax.ShapeDtypeStruct((seq, 8, n_wb, 8, CHUNK), jnp.float32),
        mesh=mesh,
        scratch_types=(
            [pltpu.VMEM((n_sb, 8, CHUNK), jnp.int32)]
            + [pltpu.VMEM((CHUNK, D_MODEL), jnp.float32)] * 4
            + [pltpu.VMEM((8, 8, 131), jnp.float32)] * 4
            + [pltpu.SemaphoreType.DMA] * 8
        ),
        compiler_params=pltpu.CompilerParams(
            use_tc_tiling_on_sc=False, needs_layout_passes=False
        ),
    )
    out4 = grid_fn(x4, tlin)
    out = out4.transpose(2, 4, 0, 1, 3).reshape(batch, seq, D_MODEL)
    return out


# speedup vs baseline: 1.2997x; 1.2997x over previous
import functools
import math

import jax
import jax.numpy as jnp
from jax import lax
from jax.experimental import pallas as pl
from jax.experimental.pallas import tpu as pltpu
from jax.experimental.pallas import tpu_sc as plsc

D_MODEL = 64
SCALE = math.sqrt(D_MODEL)
NUM_CORES = 2
NUM_SUBCORES = 16
NUM_WORKERS = NUM_CORES * NUM_SUBCORES
LANES = 16
CHUNK = 128


def _prep_body(n_blocks, tail_off, tail_n, tT_hbm, tail_hbm, tlin_hbm,
               tb0, tb1, tb2, tb3, to0, to1, to2, to3, tlv,
               gsem0, gsem1, gsem2, gsem3, ssem0, ssem1, ssem2, ssem3):
    tbuf = (tb0, tb1, tb2, tb3)
    tout = (to0, to1, to2, to3)
    gsem = (gsem0, gsem1, gsem2, gsem3)
    ssem = (ssem0, ssem1, ssem2, ssem3)

    wid = lax.axis_index("s") * NUM_CORES + lax.axis_index("c")
    nb_w = n_blocks // NUM_WORKERS
    n_left = n_blocks - nb_w * NUM_WORKERS
    base = wid * nb_w

    iota = lax.broadcasted_iota(jnp.int32, (LANES,), 0)
    lane_mask = jnp.full((LANES,), LANES - 1, jnp.int32)

    def start_load(vb, par):
        pltpu.async_copy(
            tT_hbm.at[:, pl.ds(vb * CHUNK, CHUNK)], tbuf[par], gsem[par]
        )

    def wait_load(par):
        pltpu.make_async_copy(
            tT_hbm.at[:, pl.ds(0, CHUNK)], tbuf[par], gsem[par]
        ).wait()

    def start_store(vb, par):
        pltpu.async_copy(
            tout[par], tlin_hbm.at[pl.ds(vb * CHUNK * D_MODEL,
                                         CHUNK * D_MODEL)], ssem[par]
        )

    def wait_store(par):
        pltpu.make_async_copy(
            tout[par], tlin_hbm.at[pl.ds(0, CHUNK * D_MODEL)], ssem[par]
        ).wait()

    def transpose(par):
        @plsc.parallel_loop(0, LANES, unroll=4)
        def _(d):
            rot_d = (iota + d) & lane_mask
            dout_d = rot_d * D_MODEL + iota
            for f0b in range(D_MODEL // LANES):
                rowv = iota + (f0b * LANES)
                for j0b in range(CHUNK // LANES):
                    colv = rot_d + (j0b * LANES)
                    vals = plsc.load_gather(tbuf[par], [rowv, colv])
                    idx_out = dout_d + (j0b * LANES * D_MODEL + f0b * LANES)
                    plsc.store_scatter(tout[par], [idx_out], vals)

    for p in range(4):
        start_load(base + p, p)

    @pl.loop(0, nb_w, step=4)
    def _(c):
        for p in range(4):
            cc = c + p
            wait_load(p)

            @pl.when(cc >= 4)
            def _():
                wait_store(p)

            transpose(p)
            start_store(base + cc, p)

            @pl.when(cc + 4 < nb_w)
            def _():
                start_load(base + cc + 4, p)

    for p in range(4):
        wait_store(p)

    @pl.when(wid < n_left)
    def _():
        vb = nb_w * NUM_WORKERS + wid
        pltpu.sync_copy(tT_hbm.at[:, pl.ds(vb * CHUNK, CHUNK)], tbuf[0])
        transpose(0)
        start_store(vb, 0)
        wait_store(0)

    @pl.when(wid == n_left)
    def _():
        pltpu.sync_copy(tail_hbm, tlv)
        pltpu.sync_copy(
            tlv, tlin_hbm.at[pl.ds(tail_off * D_MODEL, tail_n * D_MODEL)]
        )


def _emb_body(n_sb, x4_hbm, tp_hbm, out_hbm, idx_v, r0, r1, r2, r3,
              b0, b1, b2, b3,
              gsem0, gsem1, gsem2, gsem3, ssem0, ssem1, ssem2, ssem3):
    rbuf = (r0, r1, r2, r3)
    bbuf = (b0, b1, b2, b3)
    gsem = (gsem0, gsem1, gsem2, gsem3)
    ssem = (ssem0, ssem1, ssem2, ssem3)

    wid = lax.axis_index("s") * NUM_CORES + lax.axis_index("c")
    pltpu.sync_copy(x4_hbm.at[:, wid], idx_v)

    iota = lax.broadcasted_iota(jnp.int32, (LANES,), 0)

    def start_gather(c, par):
        pltpu.async_copy(
            tp_hbm.at[idx_v.at[c // 8, c % 8]], rbuf[par], gsem[par]
        )

    def wait_gather(par):
        pltpu.make_async_copy(
            tp_hbm.at[idx_v.at[0, 0]], rbuf[par], gsem[par]
        ).wait()

    def start_store(c, par):
        pltpu.async_copy(
            bbuf[par].at[:, :, pl.ds(0, CHUNK)], out_hbm.at[c, :, wid],
            ssem[par],
        )

    def wait_store(par):
        pltpu.make_async_copy(
            bbuf[par].at[:, :, pl.ds(0, CHUNK)], out_hbm.at[0, :, wid],
            ssem[par],
        ).wait()

    fr_vec = lax.rem(iota, jnp.full((LANES,), 8, jnp.int32))
    fb_vecs = [
        lax.div(iota, jnp.full((LANES,), 8, jnp.int32)) + (2 * g)
        for g in range(4)
    ]

    def transpose_scale(par):
        @plsc.parallel_loop(0, CHUNK, unroll=4)
        def _(j):
            jv = jnp.zeros((LANES,), jnp.int32) + j
            for g in range(4):
                vals = rbuf[par][j, pl.ds(g * LANES, LANES)] * SCALE
                plsc.store_scatter(bbuf[par], [fb_vecs[g], fr_vec, jv], vals)

    n_chunks = n_sb * 8
    for p in range(4):
        start_gather(p, p)

    @pl.loop(0, n_chunks, step=4)
    def _(c):
        for p in range(4):
            cc = c + p
            wait_gather(p)

            @pl.when(cc >= 4)
            def _():
                wait_store(p)

            transpose_scale(p)
            start_store(cc, p)

            @pl.when(cc + 4 < n_chunks)
            def _():
                start_gather(cc + 4, p)

    for p in range(4):
        wait_store(p)


def kernel(x, table):
    batch, seq = x.shape
    vocab, d_model = table.shape
    assert d_model == D_MODEL and seq % 8 == 0 and batch % CHUNK == 0
    n_sb = seq // 8
    n_wb = batch // CHUNK
    assert n_wb == NUM_WORKERS

    x4 = x.T.reshape(n_sb, 8, n_wb, CHUNK).transpose(0, 2, 1, 3)

    mesh = plsc.VectorSubcoreMesh(
        core_axis_name="c",
        subcore_axis_name="s",
        num_cores=NUM_CORES,
        num_subcores=NUM_SUBCORES,
    )

    n_blocks = vocab // CHUNK
    tail_off = n_blocks * CHUNK
    tail_n = vocab - tail_off
    tT = table.T
    tail = table[tail_off:].reshape(tail_n * D_MODEL)
    prep_fn = pl.kernel(
        functools.partial(_prep_body, n_blocks, tail_off, tail_n),
        out_type=jax.ShapeDtypeStruct((vocab * D_MODEL,), jnp.float32),
        mesh=mesh,
        scratch_types=(
            [pltpu.VMEM((D_MODEL, CHUNK), jnp.float32)] * 4
            + [pltpu.VMEM((CHUNK * D_MODEL,), jnp.float32)] * 4
            + [pltpu.VMEM((tail_n * D_MODEL,), jnp.float32)]
            + [pltpu.SemaphoreType.DMA] * 8
        ),
        compiler_params=pltpu.CompilerParams(needs_layout_passes=False),
    )
    tlin = prep_fn(tT, tail).reshape(vocab, D_MODEL)
    grid_fn = pl.kernel(
        functools.partial(_emb_body, n_sb),
        out_type=jax.ShapeDtypeStruct((seq, 8, n_wb, 8, CHUNK), jnp.float32),
        mesh=mesh,
        scratch_types=(
            [pltpu.VMEM((n_sb, 8, CHUNK), jnp.int32)]
            + [pltpu.VMEM((CHUNK, D_MODEL), jnp.float32)] * 4
            + [pltpu.VMEM((8, 8, 131), jnp.float32)] * 4
            + [pltpu.SemaphoreType.DMA] * 8
        ),
        compiler_params=pltpu.CompilerParams(
            use_tc_tiling_on_sc=False, needs_layout_passes=False
        ),
    )
    out4 = grid_fn(x4, tlin)
    out = out4.transpose(2, 4, 0, 1, 3).reshape(batch, seq, D_MODEL)
    return out
